# Initial kernel scaffold; baseline (speedup 1.0000x reference)
import jax
import jax.numpy as jnp
from jax.experimental import pallas as pl


def kernel(feats, edge_index, W_seq, b_seq, W1, b1, W2, b2, W3, b3, bn_g, bn_b, ln_g, ln_b, W_r, b_r, W_p, b_p):
    raise NotImplementedError("write your pallas kernel here")


# speedup vs baseline: 14.8419x; 14.8419x over previous
import functools

import jax
import jax.numpy as jnp
from jax import lax
from jax.experimental import pallas as pl
from jax.experimental.pallas import tpu as pltpu
from jax.experimental.pallas import tpu_sc as plsc

_N = 50000
_E = 800000
_R = 64
_NPG = 5
_NC = 10

_NW = 32
_EPW = _E // _NW
_CHUNK = 3136
_ACC = 16 * _CHUNK

_BR = 2000
_NB = _N // _BR


def _sc_body(feats_hbm, src_hbm, dst_hbm, s_out, d_out,
             idx_s, idx_d, vals, ones_v, zeros_v, s_acc, d_acc, sem):
    c = lax.axis_index("c")
    s = lax.axis_index("s")
    wid = c * 16 + s

    def _fill_zeros(i, carry):
        zeros_v[pl.ds(i * 16, 16)] = jnp.zeros((16,), jnp.float32)
        return carry

    lax.fori_loop(0, _CHUNK // 16, _fill_zeros, 0)

    def _fill_ones(i, carry):
        ones_v[pl.ds(i * 16, 16)] = jnp.ones((16,), jnp.float32)
        return carry

    lax.fori_loop(0, _EPW // 16, _fill_ones, 0)
    ones_v[pl.ds(_EPW - 16, 16)] = jnp.ones((16,), jnp.float32)

    pltpu.sync_copy(zeros_v, s_acc.at[pl.ds(s * _CHUNK, _CHUNK)])
    pltpu.sync_copy(zeros_v, d_acc.at[pl.ds(s * _CHUNK, _CHUNK)])
    plsc.subcore_barrier()

    base = wid * _EPW
    pltpu.sync_copy(src_hbm.at[pl.ds(base, _EPW)], idx_s)
    pltpu.sync_copy(dst_hbm.at[pl.ds(base, _EPW)], idx_d)
    pltpu.async_copy(feats_hbm.at[idx_s], vals, sem).wait()
    pltpu.sync_copy(vals, s_acc.at[idx_d], add=True)
    pltpu.sync_copy(ones_v, d_acc.at[idx_d], add=True)
    plsc.subcore_barrier()

    off = c * _ACC + s * _CHUNK
    pltpu.sync_copy(s_acc.at[pl.ds(s * _CHUNK, _CHUNK)], zeros_v)
    pltpu.sync_copy(zeros_v, s_out.at[pl.ds(off, _CHUNK)])
    pltpu.sync_copy(d_acc.at[pl.ds(s * _CHUNK, _CHUNK)], zeros_v)
    pltpu.sync_copy(zeros_v, d_out.at[pl.ds(off, _CHUNK)])


@functools.lru_cache(maxsize=1)
def _build_sc_segsum():
    return pl.kernel(
        _sc_body,
        out_type=(
            jax.ShapeDtypeStruct((2 * _ACC,), jnp.float32),
            jax.ShapeDtypeStruct((2 * _ACC,), jnp.float32),
        ),
        mesh=plsc.VectorSubcoreMesh(core_axis_name="c", subcore_axis_name="s"),
        scratch_types=[
            pltpu.VMEM((_EPW,), jnp.int32),
            pltpu.VMEM((_EPW,), jnp.int32),
            pltpu.VMEM((_EPW,), jnp.float32),
            pltpu.VMEM((_EPW,), jnp.float32),
            pltpu.VMEM((_CHUNK,), jnp.float32),
            pltpu.VMEM_SHARED((_ACC,), jnp.float32),
            pltpu.VMEM_SHARED((_ACC,), jnp.float32),
            pltpu.SemaphoreType.DMA,
        ],
    )


def _tc_body(feats_ref, s_ref, d_ref, w_ref, bseq_ref, W1_ref, b1_ref,
             W2_ref, b2_ref, W3_ref, b3_ref, bng_ref, bnb_ref, lng_ref,
             lnb_ref, Wr_ref, br_ref, Wp_ref, bp_ref, y_ref, h3_buf, stats):
    p = pl.program_id(0)
    i = pl.program_id(1)

    @pl.when(p == 0)
    def _phase0():
        a = feats_ref[...] + s_ref[0] + s_ref[1]
        cdeg = 1.0 + d_ref[0] + d_ref[1]
        h = a * w_ref[...] + cdeg * bseq_ref[...]
        h = jnp.maximum(
            jnp.dot(h, W1_ref[...], preferred_element_type=jnp.float32)
            + b1_ref[...], 0.0)
        h = jnp.maximum(
            jnp.dot(h, W2_ref[...], preferred_element_type=jnp.float32)
            + b2_ref[...], 0.0)
        h = jnp.maximum(
            jnp.dot(h, W3_ref[...], preferred_element_type=jnp.float32)
            + b3_ref[...], 0.0)
        h3_buf[pl.ds(i * _BR, _BR), :] = h
        colsum = jnp.sum(h, axis=0, keepdims=True)
        colsq = jnp.sum(h * h, axis=0, keepdims=True)

        @pl.when(i == 0)
        def _init():
            stats[0:1, :] = colsum
            stats[1:2, :] = colsq

        @pl.when(i > 0)
        def _accum():
            stats[0:1, :] = stats[0:1, :] + colsum
            stats[1:2, :] = stats[1:2, :] + colsq

    @pl.when(p == 1)
    def _phase1():
        ninv = jnp.float32(1.0 / _N)
        mean = stats[0:1, :] * ninv
        var = stats[1:2, :] * ninv - mean * mean
        h = h3_buf[pl.ds(i * _BR, _BR), :]
        h = (h - mean) * lax.rsqrt(var + 1e-5) * bng_ref[...] + bnb_ref[...]
        mu = jnp.mean(h, axis=1, keepdims=True)
        v = jnp.mean(h * h, axis=1, keepdims=True) - mu * mu
        h = (h - mu) * lax.rsqrt(v + 1e-5) * lng_ref[...] + lnb_ref[...]
        h = jnp.maximum(h, 0.0)
        h = jnp.dot(h, Wr_ref[...], preferred_element_type=jnp.float32) + br_ref[...]
        rowk = lax.broadcasted_iota(jnp.int32, (_BR, 1), 0) % _NPG
        z = jnp.zeros((_BR, _NC), jnp.float32)
        for k in range(_NPG):
            zk = jnp.dot(h, Wp_ref[k], preferred_element_type=jnp.float32)
            z = z + jnp.where(rowk == k, zk, 0.0)
        gi = lax.broadcasted_iota(jnp.int32, (_BR // _NPG, _BR), 0)
        ni = lax.broadcasted_iota(jnp.int32, (_BR // _NPG, _BR), 1) // _NPG
        sel = (gi == ni).astype(jnp.float32)
        y = jnp.dot(sel, z, preferred_element_type=jnp.float32) + bp_ref[...]
        y_ref[...] = y


def _row(v):
    return v.reshape(1, -1)


def kernel(feats, edge_index, W_seq, b_seq, W1, b1, W2, b2, W3, b3,
           bn_g, bn_b, ln_g, ln_b, W_r, b_r, W_p, b_p):
    feats = feats.astype(jnp.float32)
    src = edge_index[0].astype(jnp.int32)
    dst = edge_index[1].astype(jnp.int32)

    s_flat, d_flat = _build_sc_segsum()(feats, src, dst)
    s_par = s_flat.reshape(2, _ACC)[:, :_N].reshape(2, _N, 1)
    d_par = d_flat.reshape(2, _ACC)[:, :_N].reshape(2, _N, 1)

    feats_r = feats.reshape(_N, 1)
    w_row = W_seq.reshape(1, _R)
    Wp_t = W_p.reshape(_NC, _NPG, _R).transpose(1, 2, 0)

    full = lambda *bs: pl.BlockSpec(bs, lambda p, i: tuple(0 for _ in bs))
    in_specs = [
            pl.BlockSpec((_BR, 1), lambda p, i: (i, 0)),
            pl.BlockSpec((2, _BR, 1), lambda p, i: (0, i, 0)),
            pl.BlockSpec((2, _BR, 1), lambda p, i: (0, i, 0)),
            full(1, _R),
            full(1, _R),
            full(_R, _R),
            full(1, _R),
            full(_R, _R),
            full(1, _R),
            full(_R, _R),
            full(1, _R),
            full(1, _R),
            full(1, _R),
            full(1, _R),
            full(1, _R),
            full(_R, _R),
            full(1, _R),
            full(_NPG, _R, _NC),
            full(1, _NC),
    ]
    y = pl.pallas_call(
        _tc_body,
        grid=(2, _NB),
        in_specs=in_specs,
        out_specs=pl.BlockSpec((_BR // _NPG, _NC), lambda p, i: (i, 0)),
        out_shape=jax.ShapeDtypeStruct((_N // _NPG, _NC), jnp.float32),
        scratch_shapes=[
            pltpu.VMEM((_N, _R), jnp.float32),
            pltpu.VMEM((8, _R), jnp.float32),
        ],
    )(feats_r, s_par, d_par, w_row, _row(b_seq), W1.T, _row(b1), W2.T,
      _row(b2), W3.T, _row(b3), _row(bn_g), _row(bn_b), _row(ln_g),
      _row(ln_b), W_r.T, _row(b_r), Wp_t, _row(b_p))
    return y
